# baseline (speedup 1.0000x reference)
import functools

import jax
import jax.numpy as jnp
from jax import lax
from jax.experimental import pallas as pl
from jax.experimental.pallas import tpu as pltpu
from jax.experimental.pallas import tpu_sc as plsc

NC = 2
NS = 16
NW = NC * NS
CH = 80
NUM_GRAPHS = 128

_HI = jax.lax.Precision.HIGHEST


def _make_deg_kernel(E, N):
    epw = E // NW
    mesh = plsc.VectorSubcoreMesh(core_axis_name="c", subcore_axis_name="s")

    @functools.partial(
        pl.kernel,
        out_type=jax.ShapeDtypeStruct((NW * N,), jnp.float32),
        mesh=mesh,
        compiler_params=pltpu.CompilerParams(needs_layout_passes=False),
        scratch_types=[
            pltpu.VMEM((epw,), jnp.int32),
            pltpu.VMEM((N,), jnp.float32),
        ],
    )
    def deg_kernel(dst_hbm, out_hbm, dbuf, hist):
        c = lax.axis_index("c")
        s = lax.axis_index("s")
        wid = c * NS + s
        pltpu.sync_copy(dst_hbm.at[pl.ds(wid * epw, epw)], dbuf)

        zeros16 = jnp.zeros((16,), jnp.float32)

        def zbody(i, carry):
            hist[pl.ds(i * 16, 16)] = zeros16
            return carry

        lax.fori_loop(0, N // 16, zbody, 0)

        ones16 = jnp.full((16,), 1.0, jnp.float32)

        def body(i, carry):
            idx = dbuf[pl.ds(i * 16, 16)]
            plsc.addupdate_scatter(hist, [idx], ones16)
            return carry

        lax.fori_loop(0, epw // 16, body, 0)
        pltpu.sync_copy(hist, out_hbm.at[pl.ds(wid * N, N)])

    return deg_kernel


def _make_edge_kernel(E, N, D, nch, ch):
    nph = 4
    pch = nch // nph
    ngrp = pch // 4
    assert pch % 4 == 0 and (pch * 4) % 8 == 0
    rpw = (N // NS) // 8 * 8
    tail = N - NS * rpw
    mesh = plsc.VectorSubcoreMesh(core_axis_name="c", subcore_axis_name="s")

    @functools.partial(
        pl.kernel,
        out_type=jax.ShapeDtypeStruct((NC, N, D), jnp.float32),
        mesh=mesh,
        scratch_types=[
            pltpu.VMEM((pch, ch), jnp.int32),
            pltpu.VMEM((pch, ch), jnp.int32),
            [pltpu.VMEM((ch, D), jnp.float32) for _ in range(4)],
            pltpu.VMEM_SHARED((N + 8, D), jnp.float32),
            [pltpu.SemaphoreType.DMA for _ in range(4)],
            pltpu.SemaphoreType.DMA,
            pltpu.SemaphoreType.DMA,
        ],
    )
    def edge_kernel(t_hbm, src_hbm, dst_hbm, z_hbm, out_hbm,
                    sidx, didx, rows, acc, gsem, isem, zsem):
        c = lax.axis_index("c")
        s = lax.axis_index("s")
        wid = c * NS + s
        row0 = s * rpw

        def stage_idx(q):
            pltpu.async_copy(src_hbm.at[wid, pl.ds(q * pch, pch)], sidx,
                             isem)
            pltpu.async_copy(dst_hbm.at[wid, pl.ds(q * pch, pch)], didx,
                             isem)

        def wait_idx(q):
            pltpu.make_async_copy(src_hbm.at[wid, pl.ds(q * pch, pch)],
                                  sidx, isem).wait()
            pltpu.make_async_copy(dst_hbm.at[wid, pl.ds(q * pch, pch)],
                                  didx, isem).wait()

        pltpu.async_copy(z_hbm.at[pl.ds(row0, rpw)],
                         acc.at[pl.ds(row0, rpw)], zsem)
        if tail:
            @pl.when(s == NS - 1)
            def _():
                pltpu.async_copy(z_hbm.at[pl.ds(NS * rpw, tail)],
                                 acc.at[pl.ds(NS * rpw, tail)], zsem)
        stage_idx(0)
        wait_idx(0)
        for b in range(4):
            pltpu.async_copy(t_hbm.at[sidx.at[b]], rows[b], gsem[b])
        pltpu.make_async_copy(z_hbm.at[pl.ds(row0, rpw)],
                              acc.at[pl.ds(row0, rpw)], zsem).wait()
        if tail:
            @pl.when(s == NS - 1)
            def _():
                pltpu.make_async_copy(z_hbm.at[pl.ds(NS * rpw, tail)],
                                      acc.at[pl.ds(NS * rpw, tail)],
                                      zsem).wait()
        plsc.subcore_barrier()

        for q in range(nph):
            if q > 0:
                wait_idx(q)
                for b in range(4):
                    pltpu.async_copy(t_hbm.at[sidx.at[b]], rows[b], gsem[b])

            def body(jj, carry):
                for b in range(4):
                    j = jj * 4 + b
                    pltpu.make_async_copy(t_hbm.at[sidx.at[j]], rows[b],
                                          gsem[b]).wait()
                    pltpu.sync_copy(rows[b], acc.at[didx.at[j]], add=True)

                    @pl.when(jj + 1 < ngrp)
                    def _():
                        pltpu.async_copy(t_hbm.at[sidx.at[j + 4]], rows[b],
                                         gsem[b])
                return carry

            lax.fori_loop(0, ngrp, body, 0)
            if q + 1 < nph:
                stage_idx(q + 1)
        plsc.subcore_barrier()
        pltpu.sync_copy(acc.at[pl.ds(row0, rpw)],
                        out_hbm.at[c, pl.ds(row0, rpw)])
        if tail:
            @pl.when(s == NS - 1)
            def _():
                pltpu.sync_copy(acc.at[pl.ds(NS * rpw, tail)],
                                out_hbm.at[c, pl.ds(NS * rpw, tail)])

    return edge_kernel


BS = 2000

def _row_spec():
    return pl.BlockSpec((BS, 128), lambda i: (i, 0))


def _full_spec(shape):
    nd = len(shape)
    return pl.BlockSpec(shape, lambda i: (0,) * nd)


def _pre_body(x_ref, pw_ref, pb_ref, w0_ref, degp_ref,
              xw0_ref, dinv_ref, t0_ref):
    h = jnp.dot(x_ref[...], pw_ref[...], precision=_HI) + pb_ref[...]
    xw0 = jnp.dot(h, w0_ref[...], precision=_HI)
    xw0_ref[...] = xw0
    deg = jnp.sum(degp_ref[...], axis=0) + 1.0
    dinv = lax.rsqrt(deg)[:, None]
    dinv_ref[...] = dinv
    t0_ref[...] = dinv * xw0


def _layer_norm_relu(pre, g, beta):
    mu = jnp.mean(pre, axis=-1, keepdims=True)
    var = jnp.mean(jnp.square(pre - mu), axis=-1, keepdims=True)
    o = (pre - mu) * lax.rsqrt(var + 1e-5) * g + beta
    return jnp.maximum(o, 0.0)


def _make_cell_body(has_hsum_in, want_hsum_out):
    def body(*refs):
        (agg_ref, xw_ref, dinv_ref, b_ref, g_ref, beta_ref) = refs[:6]
        pos = 6
        if has_hsum_in:
            hin_ref = refs[pos]
            pos += 1
        wn_ref = refs[pos]
        pos += 1
        t_ref, xwn_ref = refs[pos], refs[pos + 1]
        pos += 2
        if want_hsum_out:
            hout_ref = refs[pos]

        dinv = dinv_ref[...]
        agg = agg_ref[...]
        a = agg[0] + agg[1]
        pre = dinv * a + (dinv * dinv) * xw_ref[...] + b_ref[...]
        o = _layer_norm_relu(pre, g_ref[...], beta_ref[...])
        if has_hsum_in:
            hsum = hin_ref[...] + o
        else:
            hsum = o
        if want_hsum_out:
            hout_ref[...] = hsum
        xwn = jnp.dot(hsum, wn_ref[...], precision=_HI)
        xwn_ref[...] = xwn
        t_ref[...] = dinv * xwn

    return body


def _final_body(agg_ref, xw_ref, dinv_ref, b_ref, g_ref, beta_ref,
                batch_ref, pw_ref, pb_ref, out_ref, pool_acc):
    i = pl.program_id(0)
    dinv = dinv_ref[...]
    agg = agg_ref[...]
    a = agg[0] + agg[1]
    pre = dinv * a + (dinv * dinv) * xw_ref[...] + b_ref[...]
    o = _layer_norm_relu(pre, g_ref[...], beta_ref[...])
    gid = lax.broadcasted_iota(jnp.int32, (1, NUM_GRAPHS), 1)
    onehot = (batch_ref[...] == gid).astype(jnp.float32)
    pooled = lax.dot_general(onehot, o, (((0,), (0,)), ((), ())),
                             precision=_HI)

    @pl.when(i == 0)
    def _():
        pool_acc[...] = jnp.zeros_like(pool_acc)

    pool_acc[...] += pooled

    @pl.when(i == pl.num_programs(0) - 1)
    def _():
        out_ref[...] = (jnp.dot(pool_acc[...], pw_ref[...], precision=_HI)
                        + pb_ref[...])


def _tc(body, grid, in_specs, out_specs, out_shape, *args, scratch_shapes=()):
    return pl.pallas_call(
        body, grid=grid, in_specs=in_specs, out_specs=out_specs,
        out_shape=out_shape, scratch_shapes=scratch_shapes,
        compiler_params=pltpu.CompilerParams(
            vmem_limit_bytes=60 * 1024 * 1024),
    )(*args)


def kernel(x, edge_index, batch, params):
    N, D = x.shape
    E = edge_index.shape[1]
    ch = 80
    epw = E // NW
    nch = -(-epw // (ch * 16)) * 16
    pad = nch * ch - epw
    src = jnp.pad(edge_index[0].reshape(NW, epw),
                  ((0, 0), (0, pad))).reshape(NW, nch, ch)
    dst_flat = edge_index[1]
    dst = jnp.pad(dst_flat.reshape(NW, epw), ((0, 0), (0, pad)),
                  constant_values=N).reshape(NW, nch, ch)
    zeros_nd = jnp.zeros((N, D), jnp.float32)
    batch2 = batch.reshape(N, 1)
    cells = params["cells"]
    ncells = len(cells)

    deg_kernel = _make_deg_kernel(E, N)
    edge_kernel = _make_edge_kernel(E, N, D, nch, ch)

    deg_p = deg_kernel(dst_flat).reshape(NW, N)
    grid = (N // BS,)
    row = _row_spec
    dinv_spec = pl.BlockSpec((BS, 1), lambda i: (i, 0))
    w_spec = _full_spec((D, D))
    v_spec = _full_spec((D,))
    agg_spec = pl.BlockSpec((NC, BS, D), lambda i: (0, i, 0))
    nd_sds = jax.ShapeDtypeStruct((N, D), jnp.float32)

    xw, dinv, t = pl.pallas_call(
        _pre_body,
        out_shape=(nd_sds, jax.ShapeDtypeStruct((N, 1), jnp.float32),
                   nd_sds),
        compiler_params=pltpu.CompilerParams(
            vmem_limit_bytes=60 * 1024 * 1024),
    )(x, params["pre_W"], params["pre_b"], cells[0]["W"], deg_p)

    hsum = None
    for i in range(ncells):
        agg = edge_kernel(t, src, dst, zeros_nd)
        c = cells[i]
        if i < ncells - 1:
            has_hin = i > 0
            want_hout = i < ncells - 2
            ins = [agg, xw, dinv, c["b"], c["g"], c["beta"]]
            specs = [agg_spec, row(), dinv_spec, v_spec, v_spec, v_spec]
            if has_hin:
                ins.append(hsum)
                specs.append(row())
            ins.append(cells[i + 1]["W"])
            specs.append(w_spec)
            outs = [nd_sds, nd_sds]
            out_specs = [row(), row()]
            if want_hout:
                outs.append(nd_sds)
                out_specs.append(row())
            res = _tc(_make_cell_body(has_hin, want_hout), grid,
                      specs, tuple(out_specs), tuple(outs), *ins)
            if want_hout:
                t, xw, hsum = res
            else:
                t, xw = res
        else:
            n_out = params["post_W"].shape[1]
            out = _tc(
                _final_body, grid,
                [agg_spec, row(), dinv_spec, v_spec, v_spec, v_spec,
                 pl.BlockSpec((BS, 1), lambda i: (i, 0)),
                 _full_spec((D, n_out)), _full_spec((n_out,))],
                pl.BlockSpec((NUM_GRAPHS, n_out), lambda i: (0, 0)),
                jax.ShapeDtypeStruct((NUM_GRAPHS, n_out), jnp.float32),
                agg, xw, dinv, c["b"], c["g"], c["beta"], batch2,
                params["post_W"], params["post_b"],
                scratch_shapes=[pltpu.VMEM((NUM_GRAPHS, D), jnp.float32)])
    return out


# speedup vs baseline: 3.0309x; 3.0309x over previous
import functools

import jax
import jax.numpy as jnp
from jax import lax
from jax.experimental import pallas as pl
from jax.experimental.pallas import tpu as pltpu
from jax.experimental.pallas import tpu_sc as plsc

NC = 2
NS = 16
NW = NC * NS
CH = 80
NUM_GRAPHS = 128

_HI = jax.lax.Precision.HIGHEST


def _make_deg_kernel(E, N):
    epw = E // NW
    mesh = plsc.VectorSubcoreMesh(core_axis_name="c", subcore_axis_name="s")

    @functools.partial(
        pl.kernel,
        out_type=jax.ShapeDtypeStruct((NW * N,), jnp.float32),
        mesh=mesh,
        compiler_params=pltpu.CompilerParams(needs_layout_passes=False),
        scratch_types=[
            pltpu.VMEM((epw,), jnp.int32),
            pltpu.VMEM((N,), jnp.float32),
        ],
    )
    def deg_kernel(dst_hbm, out_hbm, dbuf, hist):
        c = lax.axis_index("c")
        s = lax.axis_index("s")
        wid = c * NS + s
        pltpu.sync_copy(dst_hbm.at[pl.ds(wid * epw, epw)], dbuf)

        zeros16 = jnp.zeros((16,), jnp.float32)

        def zbody(i, carry):
            hist[pl.ds(i * 16, 16)] = zeros16
            return carry

        lax.fori_loop(0, N // 16, zbody, 0)

        ones16 = jnp.full((16,), 1.0, jnp.float32)

        def body(i, carry):
            idx = dbuf[pl.ds(i * 16, 16)]
            plsc.addupdate_scatter(hist, [idx], ones16)
            return carry

        lax.fori_loop(0, epw // 16, body, 0)
        pltpu.sync_copy(hist, out_hbm.at[pl.ds(wid * N, N)])

    return deg_kernel


def _make_edge_kernel(E, N, D, nch, ch):
    nph = 2
    pch = nch // nph
    assert pch % 2 == 0 and pch % 8 == 0
    rpw = (N // NS) // 8 * 8
    tail = N - NS * rpw
    mesh = plsc.VectorSubcoreMesh(core_axis_name="c", subcore_axis_name="s")

    @functools.partial(
        pl.kernel,
        out_type=jax.ShapeDtypeStruct((NC, N, D), jnp.float32),
        mesh=mesh,
        scratch_types=[
            pltpu.VMEM((pch, ch), jnp.int32),
            pltpu.VMEM((pch, ch), jnp.int32),
            pltpu.VMEM((ch, D), jnp.float32),
            pltpu.VMEM((ch, D), jnp.float32),
            pltpu.VMEM_SHARED((N + 8, D), jnp.float32),
            pltpu.SemaphoreType.DMA,
            pltpu.SemaphoreType.DMA,
            pltpu.SemaphoreType.DMA,
            pltpu.SemaphoreType.DMA,
        ],
    )
    def edge_kernel(t_hbm, src_hbm, dst_hbm, z_hbm, out_hbm,
                    sidx, didx, rows0, rows1, acc, sem0, sem1, isem, zsem):
        c = lax.axis_index("c")
        s = lax.axis_index("s")
        wid = c * NS + s
        row0 = s * rpw

        pltpu.async_copy(z_hbm.at[pl.ds(row0, rpw)],
                         acc.at[pl.ds(row0, rpw)], zsem)
        if tail:
            @pl.when(s == NS - 1)
            def _():
                pltpu.async_copy(z_hbm.at[pl.ds(NS * rpw, tail)],
                                 acc.at[pl.ds(NS * rpw, tail)], zsem)
        pltpu.async_copy(src_hbm.at[wid, pl.ds(0, pch)], sidx, isem)
        pltpu.async_copy(dst_hbm.at[wid, pl.ds(0, pch)], didx, isem)
        pltpu.make_async_copy(src_hbm.at[wid, pl.ds(0, pch)], sidx,
                              isem).wait()
        pltpu.make_async_copy(dst_hbm.at[wid, pl.ds(0, pch)], didx,
                              isem).wait()
        pltpu.async_copy(t_hbm.at[sidx.at[0]], rows0, sem0)
        pltpu.make_async_copy(z_hbm.at[pl.ds(row0, rpw)],
                              acc.at[pl.ds(row0, rpw)], zsem).wait()
        if tail:
            @pl.when(s == NS - 1)
            def _():
                pltpu.make_async_copy(z_hbm.at[pl.ds(NS * rpw, tail)],
                                      acc.at[pl.ds(NS * rpw, tail)],
                                      zsem).wait()
        plsc.subcore_barrier()

        for half in range(nph):
            if half > 0:
                pltpu.sync_copy(src_hbm.at[wid, pl.ds(half * pch, pch)],
                                sidx)
                pltpu.sync_copy(dst_hbm.at[wid, pl.ds(half * pch, pch)],
                                didx)
                pltpu.async_copy(t_hbm.at[sidx.at[0]], rows0, sem0)

            def body(jj, carry):
                j = jj * 2
                pltpu.async_copy(t_hbm.at[sidx.at[j + 1]], rows1, sem1)
                pltpu.make_async_copy(t_hbm.at[sidx.at[j]], rows0,
                                      sem0).wait()
                pltpu.sync_copy(rows0, acc.at[didx.at[j]], add=True)

                @pl.when(jj + 1 < pch // 2)
                def _():
                    pltpu.async_copy(t_hbm.at[sidx.at[j + 2]], rows0, sem0)

                pltpu.make_async_copy(t_hbm.at[sidx.at[j + 1]], rows1,
                                      sem1).wait()
                pltpu.sync_copy(rows1, acc.at[didx.at[j + 1]], add=True)
                return carry

            lax.fori_loop(0, pch // 2, body, 0)
        plsc.subcore_barrier()
        pltpu.sync_copy(acc.at[pl.ds(row0, rpw)],
                        out_hbm.at[c, pl.ds(row0, rpw)])
        if tail:
            @pl.when(s == NS - 1)
            def _():
                pltpu.sync_copy(acc.at[pl.ds(NS * rpw, tail)],
                                out_hbm.at[c, pl.ds(NS * rpw, tail)])

    return edge_kernel


BS = 2000

def _row_spec():
    return pl.BlockSpec((BS, 128), lambda i: (i, 0))


def _full_spec(shape):
    nd = len(shape)
    return pl.BlockSpec(shape, lambda i: (0,) * nd)


def _pre_body(x_ref, pw_ref, pb_ref, w0_ref, degp_ref,
              xw0_ref, dinv_ref, t0_ref):
    h = jnp.dot(x_ref[...], pw_ref[...], precision=_HI) + pb_ref[...]
    xw0 = jnp.dot(h, w0_ref[...], precision=_HI)
    xw0_ref[...] = xw0
    deg = jnp.sum(degp_ref[...], axis=0) + 1.0
    dinv = lax.rsqrt(deg)[:, None]
    dinv_ref[...] = dinv
    t0_ref[...] = dinv * xw0


def _layer_norm_relu(pre, g, beta):
    mu = jnp.mean(pre, axis=-1, keepdims=True)
    var = jnp.mean(jnp.square(pre - mu), axis=-1, keepdims=True)
    o = (pre - mu) * lax.rsqrt(var + 1e-5) * g + beta
    return jnp.maximum(o, 0.0)


def _make_cell_body(has_hsum_in, want_hsum_out):
    def body(*refs):
        (agg_ref, xw_ref, dinv_ref, b_ref, g_ref, beta_ref) = refs[:6]
        pos = 6
        if has_hsum_in:
            hin_ref = refs[pos]
            pos += 1
        wn_ref = refs[pos]
        pos += 1
        t_ref, xwn_ref = refs[pos], refs[pos + 1]
        pos += 2
        if want_hsum_out:
            hout_ref = refs[pos]

        dinv = dinv_ref[...]
        agg = agg_ref[...]
        a = agg[0] + agg[1]
        pre = dinv * a + (dinv * dinv) * xw_ref[...] + b_ref[...]
        o = _layer_norm_relu(pre, g_ref[...], beta_ref[...])
        if has_hsum_in:
            hsum = hin_ref[...] + o
        else:
            hsum = o
        if want_hsum_out:
            hout_ref[...] = hsum
        xwn = jnp.dot(hsum, wn_ref[...], precision=_HI)
        xwn_ref[...] = xwn
        t_ref[...] = dinv * xwn

    return body


def _final_body(agg_ref, xw_ref, dinv_ref, b_ref, g_ref, beta_ref,
                batch_ref, pw_ref, pb_ref, out_ref, pool_acc):
    i = pl.program_id(0)
    dinv = dinv_ref[...]
    agg = agg_ref[...]
    a = agg[0] + agg[1]
    pre = dinv * a + (dinv * dinv) * xw_ref[...] + b_ref[...]
    o = _layer_norm_relu(pre, g_ref[...], beta_ref[...])
    gid = lax.broadcasted_iota(jnp.int32, (1, NUM_GRAPHS), 1)
    onehot = (batch_ref[...] == gid).astype(jnp.float32)
    pooled = lax.dot_general(onehot, o, (((0,), (0,)), ((), ())),
                             precision=_HI)

    @pl.when(i == 0)
    def _():
        pool_acc[...] = jnp.zeros_like(pool_acc)

    pool_acc[...] += pooled

    @pl.when(i == pl.num_programs(0) - 1)
    def _():
        out_ref[...] = (jnp.dot(pool_acc[...], pw_ref[...], precision=_HI)
                        + pb_ref[...])


def _tc(body, grid, in_specs, out_specs, out_shape, *args, scratch_shapes=()):
    return pl.pallas_call(
        body, grid=grid, in_specs=in_specs, out_specs=out_specs,
        out_shape=out_shape, scratch_shapes=scratch_shapes,
        compiler_params=pltpu.CompilerParams(
            vmem_limit_bytes=60 * 1024 * 1024),
    )(*args)


def kernel(x, edge_index, batch, params):
    N, D = x.shape
    E = edge_index.shape[1]
    ch = 125
    epw = E // NW
    nch = -(-epw // (ch * 16)) * 16
    pad = nch * ch - epw
    src = jnp.pad(edge_index[0].reshape(NW, epw),
                  ((0, 0), (0, pad))).reshape(NW, nch, ch)
    dst_flat = edge_index[1]
    dst = jnp.pad(dst_flat.reshape(NW, epw), ((0, 0), (0, pad)),
                  constant_values=N).reshape(NW, nch, ch)
    zeros_nd = jnp.zeros((N, D), jnp.float32)
    batch2 = batch.reshape(N, 1)
    cells = params["cells"]
    ncells = len(cells)

    deg_kernel = _make_deg_kernel(E, N)
    edge_kernel = _make_edge_kernel(E, N, D, nch, ch)

    deg_p = deg_kernel(dst_flat).reshape(NW, N)
    grid = (N // BS,)
    row = _row_spec
    dinv_spec = pl.BlockSpec((BS, 1), lambda i: (i, 0))
    w_spec = _full_spec((D, D))
    v_spec = _full_spec((D,))
    agg_spec = pl.BlockSpec((NC, BS, D), lambda i: (0, i, 0))
    nd_sds = jax.ShapeDtypeStruct((N, D), jnp.float32)

    xw, dinv, t = pl.pallas_call(
        _pre_body,
        out_shape=(nd_sds, jax.ShapeDtypeStruct((N, 1), jnp.float32),
                   nd_sds),
        compiler_params=pltpu.CompilerParams(
            vmem_limit_bytes=60 * 1024 * 1024),
    )(x, params["pre_W"], params["pre_b"], cells[0]["W"], deg_p)

    hsum = None
    for i in range(ncells):
        agg = edge_kernel(t, src, dst, zeros_nd)
        c = cells[i]
        if i < ncells - 1:
            has_hin = i > 0
            want_hout = i < ncells - 2
            ins = [agg, xw, dinv, c["b"], c["g"], c["beta"]]
            specs = [agg_spec, row(), dinv_spec, v_spec, v_spec, v_spec]
            if has_hin:
                ins.append(hsum)
                specs.append(row())
            ins.append(cells[i + 1]["W"])
            specs.append(w_spec)
            outs = [nd_sds, nd_sds]
            out_specs = [row(), row()]
            if want_hout:
                outs.append(nd_sds)
                out_specs.append(row())
            res = _tc(_make_cell_body(has_hin, want_hout), grid,
                      specs, tuple(out_specs), tuple(outs), *ins)
            if want_hout:
                t, xw, hsum = res
            else:
                t, xw = res
        else:
            n_out = params["post_W"].shape[1]
            out = _tc(
                _final_body, grid,
                [agg_spec, row(), dinv_spec, v_spec, v_spec, v_spec,
                 pl.BlockSpec((BS, 1), lambda i: (i, 0)),
                 _full_spec((D, n_out)), _full_spec((n_out,))],
                pl.BlockSpec((NUM_GRAPHS, n_out), lambda i: (0, 0)),
                jax.ShapeDtypeStruct((NUM_GRAPHS, n_out), jnp.float32),
                agg, xw, dinv, c["b"], c["g"], c["beta"], batch2,
                params["post_W"], params["post_b"],
                scratch_shapes=[pltpu.VMEM((NUM_GRAPHS, D), jnp.float32)])
    return out


# speedup vs baseline: 3.0366x; 1.0019x over previous
import functools

import jax
import jax.numpy as jnp
from jax import lax
from jax.experimental import pallas as pl
from jax.experimental.pallas import tpu as pltpu
from jax.experimental.pallas import tpu_sc as plsc

NC = 2
NS = 16
NW = NC * NS
CH = 80
NUM_GRAPHS = 128

_HI = jax.lax.Precision.HIGHEST


def _make_deg_kernel(E, N):
    epw = E // NW
    mesh = plsc.VectorSubcoreMesh(core_axis_name="c", subcore_axis_name="s")

    @functools.partial(
        pl.kernel,
        out_type=jax.ShapeDtypeStruct((NW * N,), jnp.float32),
        mesh=mesh,
        compiler_params=pltpu.CompilerParams(needs_layout_passes=False),
        scratch_types=[
            pltpu.VMEM((epw,), jnp.int32),
            pltpu.VMEM((N,), jnp.float32),
        ],
    )
    def deg_kernel(dst_hbm, out_hbm, dbuf, hist):
        c = lax.axis_index("c")
        s = lax.axis_index("s")
        wid = c * NS + s
        pltpu.sync_copy(dst_hbm.at[pl.ds(wid * epw, epw)], dbuf)

        zeros16 = jnp.zeros((16,), jnp.float32)

        def zbody(i, carry):
            hist[pl.ds(i * 16, 16)] = zeros16
            return carry

        lax.fori_loop(0, N // 16, zbody, 0)

        ones16 = jnp.full((16,), 1.0, jnp.float32)

        def body(i, carry):
            idx = dbuf[pl.ds(i * 16, 16)]
            plsc.addupdate_scatter(hist, [idx], ones16)
            return carry

        lax.fori_loop(0, epw // 16, body, 0)
        pltpu.sync_copy(hist, out_hbm.at[pl.ds(wid * N, N)])

    return deg_kernel


def _make_edge_kernel(E, N, D, nch, ch):
    nph = 2
    pch = nch // nph
    assert pch % 2 == 0 and pch % 8 == 0
    rpw = (N // NS) // 8 * 8
    tail = N - NS * rpw
    mesh = plsc.VectorSubcoreMesh(core_axis_name="c", subcore_axis_name="s")

    @functools.partial(
        pl.kernel,
        out_type=jax.ShapeDtypeStruct((NC, N, D), jnp.float32),
        mesh=mesh,
        scratch_types=[
            pltpu.VMEM((pch, ch), jnp.int32),
            pltpu.VMEM((pch, ch), jnp.int32),
            pltpu.VMEM((ch, D), jnp.float32),
            pltpu.VMEM((ch, D), jnp.float32),
            pltpu.VMEM_SHARED((N + 8, D), jnp.float32),
            pltpu.SemaphoreType.DMA,
            pltpu.SemaphoreType.DMA,
            pltpu.SemaphoreType.DMA,
            pltpu.SemaphoreType.DMA,
        ],
    )
    def edge_kernel(t_hbm, src_hbm, dst_hbm, z_hbm, out_hbm,
                    sidx, didx, rows0, rows1, acc, sem0, sem1, isem, zsem):
        c = lax.axis_index("c")
        s = lax.axis_index("s")
        wid = c * NS + s
        row0 = s * rpw

        pltpu.async_copy(z_hbm.at[pl.ds(row0, rpw)],
                         acc.at[pl.ds(row0, rpw)], zsem)
        if tail:
            @pl.when(s == NS - 1)
            def _():
                pltpu.async_copy(z_hbm.at[pl.ds(NS * rpw, tail)],
                                 acc.at[pl.ds(NS * rpw, tail)], zsem)
        pltpu.async_copy(src_hbm.at[wid, pl.ds(0, pch)], sidx, isem)
        pltpu.async_copy(dst_hbm.at[wid, pl.ds(0, pch)], didx, isem)
        pltpu.make_async_copy(src_hbm.at[wid, pl.ds(0, pch)], sidx,
                              isem).wait()
        pltpu.make_async_copy(dst_hbm.at[wid, pl.ds(0, pch)], didx,
                              isem).wait()
        pltpu.async_copy(t_hbm.at[sidx.at[0]], rows0, sem0)
        pltpu.async_copy(t_hbm.at[sidx.at[1]], rows1, sem1)
        pltpu.make_async_copy(z_hbm.at[pl.ds(row0, rpw)],
                              acc.at[pl.ds(row0, rpw)], zsem).wait()
        if tail:
            @pl.when(s == NS - 1)
            def _():
                pltpu.make_async_copy(z_hbm.at[pl.ds(NS * rpw, tail)],
                                      acc.at[pl.ds(NS * rpw, tail)],
                                      zsem).wait()
        plsc.subcore_barrier()

        for half in range(nph):
            if half > 0:
                pltpu.sync_copy(src_hbm.at[wid, pl.ds(half * pch, pch)],
                                sidx)
                pltpu.sync_copy(dst_hbm.at[wid, pl.ds(half * pch, pch)],
                                didx)
                pltpu.async_copy(t_hbm.at[sidx.at[0]], rows0, sem0)
                pltpu.async_copy(t_hbm.at[sidx.at[1]], rows1, sem1)

            def body(jj, carry):
                j = jj * 2
                pltpu.make_async_copy(t_hbm.at[sidx.at[j]], rows0,
                                      sem0).wait()
                pltpu.sync_copy(rows0, acc.at[didx.at[j]], add=True)

                @pl.when(jj + 1 < pch // 2)
                def _():
                    pltpu.async_copy(t_hbm.at[sidx.at[j + 2]], rows0, sem0)

                pltpu.make_async_copy(t_hbm.at[sidx.at[j + 1]], rows1,
                                      sem1).wait()
                pltpu.sync_copy(rows1, acc.at[didx.at[j + 1]], add=True)

                @pl.when(jj + 1 < pch // 2)
                def _():
                    pltpu.async_copy(t_hbm.at[sidx.at[j + 3]], rows1, sem1)

                return carry

            lax.fori_loop(0, pch // 2, body, 0)
        plsc.subcore_barrier()
        pltpu.sync_copy(acc.at[pl.ds(row0, rpw)],
                        out_hbm.at[c, pl.ds(row0, rpw)])
        if tail:
            @pl.when(s == NS - 1)
            def _():
                pltpu.sync_copy(acc.at[pl.ds(NS * rpw, tail)],
                                out_hbm.at[c, pl.ds(NS * rpw, tail)])

    return edge_kernel


BS = 2000

def _row_spec():
    return pl.BlockSpec((BS, 128), lambda i: (i, 0))


def _full_spec(shape):
    nd = len(shape)
    return pl.BlockSpec(shape, lambda i: (0,) * nd)


def _pre_body(x_ref, pw_ref, pb_ref, w0_ref, degp_ref,
              xw0_ref, dinv_ref, t0_ref):
    h = jnp.dot(x_ref[...], pw_ref[...], precision=_HI) + pb_ref[...]
    xw0 = jnp.dot(h, w0_ref[...], precision=_HI)
    xw0_ref[...] = xw0
    deg = jnp.sum(degp_ref[...], axis=0) + 1.0
    dinv = lax.rsqrt(deg)[:, None]
    dinv_ref[...] = dinv
    t0_ref[...] = dinv * xw0


def _layer_norm_relu(pre, g, beta):
    mu = jnp.mean(pre, axis=-1, keepdims=True)
    var = jnp.mean(jnp.square(pre - mu), axis=-1, keepdims=True)
    o = (pre - mu) * lax.rsqrt(var + 1e-5) * g + beta
    return jnp.maximum(o, 0.0)


def _make_cell_body(has_hsum_in, want_hsum_out):
    def body(*refs):
        (agg_ref, xw_ref, dinv_ref, b_ref, g_ref, beta_ref) = refs[:6]
        pos = 6
        if has_hsum_in:
            hin_ref = refs[pos]
            pos += 1
        wn_ref = refs[pos]
        pos += 1
        t_ref, xwn_ref = refs[pos], refs[pos + 1]
        pos += 2
        if want_hsum_out:
            hout_ref = refs[pos]

        dinv = dinv_ref[...]
        agg = agg_ref[...]
        a = agg[0] + agg[1]
        pre = dinv * a + (dinv * dinv) * xw_ref[...] + b_ref[...]
        o = _layer_norm_relu(pre, g_ref[...], beta_ref[...])
        if has_hsum_in:
            hsum = hin_ref[...] + o
        else:
            hsum = o
        if want_hsum_out:
            hout_ref[...] = hsum
        xwn = jnp.dot(hsum, wn_ref[...], precision=_HI)
        xwn_ref[...] = xwn
        t_ref[...] = dinv * xwn

    return body


def _final_body(agg_ref, xw_ref, dinv_ref, b_ref, g_ref, beta_ref,
                batch_ref, pw_ref, pb_ref, out_ref, pool_acc):
    i = pl.program_id(0)
    dinv = dinv_ref[...]
    agg = agg_ref[...]
    a = agg[0] + agg[1]
    pre = dinv * a + (dinv * dinv) * xw_ref[...] + b_ref[...]
    o = _layer_norm_relu(pre, g_ref[...], beta_ref[...])
    gid = lax.broadcasted_iota(jnp.int32, (1, NUM_GRAPHS), 1)
    onehot = (batch_ref[...] == gid).astype(jnp.float32)
    pooled = lax.dot_general(onehot, o, (((0,), (0,)), ((), ())),
                             precision=_HI)

    @pl.when(i == 0)
    def _():
        pool_acc[...] = jnp.zeros_like(pool_acc)

    pool_acc[...] += pooled

    @pl.when(i == pl.num_programs(0) - 1)
    def _():
        out_ref[...] = (jnp.dot(pool_acc[...], pw_ref[...], precision=_HI)
                        + pb_ref[...])


def _tc(body, grid, in_specs, out_specs, out_shape, *args, scratch_shapes=()):
    return pl.pallas_call(
        body, grid=grid, in_specs=in_specs, out_specs=out_specs,
        out_shape=out_shape, scratch_shapes=scratch_shapes,
        compiler_params=pltpu.CompilerParams(
            vmem_limit_bytes=60 * 1024 * 1024),
    )(*args)


def kernel(x, edge_index, batch, params):
    N, D = x.shape
    E = edge_index.shape[1]
    ch = 125
    epw = E // NW
    nch = -(-epw // (ch * 16)) * 16
    pad = nch * ch - epw
    src = jnp.pad(edge_index[0].reshape(NW, epw),
                  ((0, 0), (0, pad))).reshape(NW, nch, ch)
    dst_flat = edge_index[1]
    dst = jnp.pad(dst_flat.reshape(NW, epw), ((0, 0), (0, pad)),
                  constant_values=N).reshape(NW, nch, ch)
    zeros_nd = jnp.zeros((N, D), jnp.float32)
    batch2 = batch.reshape(N, 1)
    cells = params["cells"]
    ncells = len(cells)

    deg_kernel = _make_deg_kernel(E, N)
    edge_kernel = _make_edge_kernel(E, N, D, nch, ch)

    deg_p = deg_kernel(dst_flat).reshape(NW, N)
    grid = (N // BS,)
    row = _row_spec
    dinv_spec = pl.BlockSpec((BS, 1), lambda i: (i, 0))
    w_spec = _full_spec((D, D))
    v_spec = _full_spec((D,))
    agg_spec = pl.BlockSpec((NC, BS, D), lambda i: (0, i, 0))
    nd_sds = jax.ShapeDtypeStruct((N, D), jnp.float32)

    xw, dinv, t = pl.pallas_call(
        _pre_body,
        out_shape=(nd_sds, jax.ShapeDtypeStruct((N, 1), jnp.float32),
                   nd_sds),
        compiler_params=pltpu.CompilerParams(
            vmem_limit_bytes=60 * 1024 * 1024),
    )(x, params["pre_W"], params["pre_b"], cells[0]["W"], deg_p)

    hsum = None
    for i in range(ncells):
        agg = edge_kernel(t, src, dst, zeros_nd)
        c = cells[i]
        if i < ncells - 1:
            has_hin = i > 0
            want_hout = i < ncells - 2
            ins = [agg, xw, dinv, c["b"], c["g"], c["beta"]]
            specs = [agg_spec, row(), dinv_spec, v_spec, v_spec, v_spec]
            if has_hin:
                ins.append(hsum)
                specs.append(row())
            ins.append(cells[i + 1]["W"])
            specs.append(w_spec)
            outs = [nd_sds, nd_sds]
            out_specs = [row(), row()]
            if want_hout:
                outs.append(nd_sds)
                out_specs.append(row())
            res = _tc(_make_cell_body(has_hin, want_hout), grid,
                      specs, tuple(out_specs), tuple(outs), *ins)
            if want_hout:
                t, xw, hsum = res
            else:
                t, xw = res
        else:
            n_out = params["post_W"].shape[1]
            out = _tc(
                _final_body, grid,
                [agg_spec, row(), dinv_spec, v_spec, v_spec, v_spec,
                 pl.BlockSpec((BS, 1), lambda i: (i, 0)),
                 _full_spec((D, n_out)), _full_spec((n_out,))],
                pl.BlockSpec((NUM_GRAPHS, n_out), lambda i: (0, 0)),
                jax.ShapeDtypeStruct((NUM_GRAPHS, n_out), jnp.float32),
                agg, xw, dinv, c["b"], c["g"], c["beta"], batch2,
                params["post_W"], params["post_b"],
                scratch_shapes=[pltpu.VMEM((NUM_GRAPHS, D), jnp.float32)])
    return out


# speedup vs baseline: 3.0455x; 1.0029x over previous
import functools

import jax
import jax.numpy as jnp
from jax import lax
from jax.experimental import pallas as pl
from jax.experimental.pallas import tpu as pltpu
from jax.experimental.pallas import tpu_sc as plsc

NC = 2
NS = 16
NW = NC * NS
CH = 80
NUM_GRAPHS = 128

_HI = jax.lax.Precision.HIGHEST


def _make_deg_kernel(E, N):
    epw = E // NW
    mesh = plsc.VectorSubcoreMesh(core_axis_name="c", subcore_axis_name="s")

    @functools.partial(
        pl.kernel,
        out_type=jax.ShapeDtypeStruct((NW * N,), jnp.float32),
        mesh=mesh,
        compiler_params=pltpu.CompilerParams(needs_layout_passes=False),
        scratch_types=[
            pltpu.VMEM((epw,), jnp.int32),
            pltpu.VMEM((N,), jnp.float32),
        ],
    )
    def deg_kernel(dst_hbm, out_hbm, dbuf, hist):
        c = lax.axis_index("c")
        s = lax.axis_index("s")
        wid = c * NS + s
        pltpu.sync_copy(dst_hbm.at[pl.ds(wid * epw, epw)], dbuf)

        zeros16 = jnp.zeros((16,), jnp.float32)

        def zbody(i, carry):
            hist[pl.ds(i * 16, 16)] = zeros16
            return carry

        lax.fori_loop(0, N // 16, zbody, 0)

        ones16 = jnp.full((16,), 1.0, jnp.float32)

        def body(i, carry):
            idx = dbuf[pl.ds(i * 16, 16)]
            plsc.addupdate_scatter(hist, [idx], ones16)
            return carry

        lax.fori_loop(0, epw // 16, body, 0)
        pltpu.sync_copy(hist, out_hbm.at[pl.ds(wid * N, N)])

    return deg_kernel


def _make_edge_kernel(E, N, D, nch, ch):
    nph = 5
    pch = nch // nph
    assert nch == nph * pch and pch % 2 == 0 and pch % 8 == 0
    rpw = (N // NS) // 8 * 8
    tail = N - NS * rpw
    mesh = plsc.VectorSubcoreMesh(core_axis_name="c", subcore_axis_name="s")

    @functools.partial(
        pl.kernel,
        out_type=jax.ShapeDtypeStruct((NC, N, D), jnp.float32),
        mesh=mesh,
        scratch_types=[
            [pltpu.VMEM((pch, ch), jnp.int32) for _ in range(2)],
            [pltpu.VMEM((pch, ch), jnp.int32) for _ in range(2)],
            pltpu.VMEM((ch, D), jnp.float32),
            pltpu.VMEM((ch, D), jnp.float32),
            pltpu.VMEM_SHARED((N + 8, D), jnp.float32),
            pltpu.SemaphoreType.DMA,
            pltpu.SemaphoreType.DMA,
            pltpu.SemaphoreType.DMA,
            pltpu.SemaphoreType.DMA,
        ],
    )
    def edge_kernel(t_hbm, src_hbm, dst_hbm, z_hbm, out_hbm,
                    sidxs, didxs, rows0, rows1, acc, sem0, sem1, isem, zsem):
        c = lax.axis_index("c")
        s = lax.axis_index("s")
        wid = c * NS + s
        row0 = s * rpw
        n2 = pch // 2

        def stage(q):
            pltpu.async_copy(src_hbm.at[wid, pl.ds(q * pch, pch)],
                             sidxs[q % 2], isem)
            pltpu.async_copy(dst_hbm.at[wid, pl.ds(q * pch, pch)],
                             didxs[q % 2], isem)

        def wait_stage(q):
            pltpu.make_async_copy(src_hbm.at[wid, pl.ds(q * pch, pch)],
                                  sidxs[q % 2], isem).wait()
            pltpu.make_async_copy(dst_hbm.at[wid, pl.ds(q * pch, pch)],
                                  didxs[q % 2], isem).wait()

        pltpu.async_copy(z_hbm.at[pl.ds(row0, rpw)],
                         acc.at[pl.ds(row0, rpw)], zsem)
        if tail:
            @pl.when(s == NS - 1)
            def _():
                pltpu.async_copy(z_hbm.at[pl.ds(NS * rpw, tail)],
                                 acc.at[pl.ds(NS * rpw, tail)], zsem)
        stage(0)
        wait_stage(0)
        pltpu.async_copy(t_hbm.at[sidxs[0].at[0]], rows0, sem0)
        pltpu.async_copy(t_hbm.at[sidxs[0].at[1]], rows1, sem1)
        pltpu.make_async_copy(z_hbm.at[pl.ds(row0, rpw)],
                              acc.at[pl.ds(row0, rpw)], zsem).wait()
        if tail:
            @pl.when(s == NS - 1)
            def _():
                pltpu.make_async_copy(z_hbm.at[pl.ds(NS * rpw, tail)],
                                      acc.at[pl.ds(NS * rpw, tail)],
                                      zsem).wait()
        plsc.subcore_barrier()

        for q in range(nph):
            sidx, didx = sidxs[q % 2], didxs[q % 2]
            nsidx = sidxs[(q + 1) % 2]
            if q + 1 < nph:
                stage(q + 1)

            def body(jj, carry):
                j = jj * 2
                pltpu.make_async_copy(t_hbm.at[sidx.at[j]], rows0,
                                      sem0).wait()
                pltpu.sync_copy(rows0, acc.at[didx.at[j]], add=True)

                @pl.when(jj + 1 < n2)
                def _():
                    pltpu.async_copy(t_hbm.at[sidx.at[j + 2]], rows0, sem0)

                if q + 1 < nph:
                    @pl.when(jj + 1 == n2)
                    def _():
                        wait_stage(q + 1)
                        pltpu.async_copy(t_hbm.at[nsidx.at[0]], rows0, sem0)

                pltpu.make_async_copy(t_hbm.at[sidx.at[j + 1]], rows1,
                                      sem1).wait()
                pltpu.sync_copy(rows1, acc.at[didx.at[j + 1]], add=True)

                @pl.when(jj + 1 < n2)
                def _():
                    pltpu.async_copy(t_hbm.at[sidx.at[j + 3]], rows1, sem1)

                if q + 1 < nph:
                    @pl.when(jj + 1 == n2)
                    def _():
                        pltpu.async_copy(t_hbm.at[nsidx.at[1]], rows1, sem1)

                return carry

            lax.fori_loop(0, n2, body, 0)
        plsc.subcore_barrier()
        pltpu.sync_copy(acc.at[pl.ds(row0, rpw)],
                        out_hbm.at[c, pl.ds(row0, rpw)])
        if tail:
            @pl.when(s == NS - 1)
            def _():
                pltpu.sync_copy(acc.at[pl.ds(NS * rpw, tail)],
                                out_hbm.at[c, pl.ds(NS * rpw, tail)])

    return edge_kernel


BS = 2000

def _row_spec():
    return pl.BlockSpec((BS, 128), lambda i: (i, 0))


def _full_spec(shape):
    nd = len(shape)
    return pl.BlockSpec(shape, lambda i: (0,) * nd)


def _pre_body(x_ref, pw_ref, pb_ref, w0_ref, degp_ref,
              xw0_ref, dinv_ref, t0_ref):
    h = jnp.dot(x_ref[...], pw_ref[...], precision=_HI) + pb_ref[...]
    xw0 = jnp.dot(h, w0_ref[...], precision=_HI)
    xw0_ref[...] = xw0
    deg = jnp.sum(degp_ref[...], axis=0) + 1.0
    dinv = lax.rsqrt(deg)[:, None]
    dinv_ref[...] = dinv
    t0_ref[...] = dinv * xw0


def _layer_norm_relu(pre, g, beta):
    mu = jnp.mean(pre, axis=-1, keepdims=True)
    var = jnp.mean(jnp.square(pre - mu), axis=-1, keepdims=True)
    o = (pre - mu) * lax.rsqrt(var + 1e-5) * g + beta
    return jnp.maximum(o, 0.0)


def _make_cell_body(has_hsum_in, want_hsum_out):
    def body(*refs):
        (agg_ref, xw_ref, dinv_ref, b_ref, g_ref, beta_ref) = refs[:6]
        pos = 6
        if has_hsum_in:
            hin_ref = refs[pos]
            pos += 1
        wn_ref = refs[pos]
        pos += 1
        t_ref, xwn_ref = refs[pos], refs[pos + 1]
        pos += 2
        if want_hsum_out:
            hout_ref = refs[pos]

        dinv = dinv_ref[...]
        agg = agg_ref[...]
        a = agg[0] + agg[1]
        pre = dinv * a + (dinv * dinv) * xw_ref[...] + b_ref[...]
        o = _layer_norm_relu(pre, g_ref[...], beta_ref[...])
        if has_hsum_in:
            hsum = hin_ref[...] + o
        else:
            hsum = o
        if want_hsum_out:
            hout_ref[...] = hsum
        xwn = jnp.dot(hsum, wn_ref[...], precision=_HI)
        xwn_ref[...] = xwn
        t_ref[...] = dinv * xwn

    return body


def _final_body(agg_ref, xw_ref, dinv_ref, b_ref, g_ref, beta_ref,
                batch_ref, pw_ref, pb_ref, out_ref, pool_acc):
    i = pl.program_id(0)
    dinv = dinv_ref[...]
    agg = agg_ref[...]
    a = agg[0] + agg[1]
    pre = dinv * a + (dinv * dinv) * xw_ref[...] + b_ref[...]
    o = _layer_norm_relu(pre, g_ref[...], beta_ref[...])
    gid = lax.broadcasted_iota(jnp.int32, (1, NUM_GRAPHS), 1)
    onehot = (batch_ref[...] == gid).astype(jnp.float32)
    pooled = lax.dot_general(onehot, o, (((0,), (0,)), ((), ())),
                             precision=_HI)

    @pl.when(i == 0)
    def _():
        pool_acc[...] = jnp.zeros_like(pool_acc)

    pool_acc[...] += pooled

    @pl.when(i == pl.num_programs(0) - 1)
    def _():
        out_ref[...] = (jnp.dot(pool_acc[...], pw_ref[...], precision=_HI)
                        + pb_ref[...])


def _tc(body, grid, in_specs, out_specs, out_shape, *args, scratch_shapes=()):
    return pl.pallas_call(
        body, grid=grid, in_specs=in_specs, out_specs=out_specs,
        out_shape=out_shape, scratch_shapes=scratch_shapes,
        compiler_params=pltpu.CompilerParams(
            vmem_limit_bytes=60 * 1024 * 1024),
    )(*args)


def kernel(x, edge_index, batch, params):
    N, D = x.shape
    E = edge_index.shape[1]
    ch = 125
    epw = E // NW
    nch = -(-epw // (ch * 16)) * 16
    pad = nch * ch - epw
    src = jnp.pad(edge_index[0].reshape(NW, epw),
                  ((0, 0), (0, pad))).reshape(NW, nch, ch)
    dst_flat = edge_index[1]
    dst = jnp.pad(dst_flat.reshape(NW, epw), ((0, 0), (0, pad)),
                  constant_values=N).reshape(NW, nch, ch)
    zeros_nd = jnp.zeros((N, D), jnp.float32)
    batch2 = batch.reshape(N, 1)
    cells = params["cells"]
    ncells = len(cells)

    deg_kernel = _make_deg_kernel(E, N)
    edge_kernel = _make_edge_kernel(E, N, D, nch, ch)

    deg_p = deg_kernel(dst_flat).reshape(NW, N)
    grid = (N // BS,)
    row = _row_spec
    dinv_spec = pl.BlockSpec((BS, 1), lambda i: (i, 0))
    w_spec = _full_spec((D, D))
    v_spec = _full_spec((D,))
    agg_spec = pl.BlockSpec((NC, BS, D), lambda i: (0, i, 0))
    nd_sds = jax.ShapeDtypeStruct((N, D), jnp.float32)

    xw, dinv, t = pl.pallas_call(
        _pre_body,
        out_shape=(nd_sds, jax.ShapeDtypeStruct((N, 1), jnp.float32),
                   nd_sds),
        compiler_params=pltpu.CompilerParams(
            vmem_limit_bytes=60 * 1024 * 1024),
    )(x, params["pre_W"], params["pre_b"], cells[0]["W"], deg_p)

    hsum = None
    for i in range(ncells):
        agg = edge_kernel(t, src, dst, zeros_nd)
        c = cells[i]
        if i < ncells - 1:
            has_hin = i > 0
            want_hout = i < ncells - 2
            ins = [agg, xw, dinv, c["b"], c["g"], c["beta"]]
            specs = [agg_spec, row(), dinv_spec, v_spec, v_spec, v_spec]
            if has_hin:
                ins.append(hsum)
                specs.append(row())
            ins.append(cells[i + 1]["W"])
            specs.append(w_spec)
            outs = [nd_sds, nd_sds]
            out_specs = [row(), row()]
            if want_hout:
                outs.append(nd_sds)
                out_specs.append(row())
            res = _tc(_make_cell_body(has_hin, want_hout), grid,
                      specs, tuple(out_specs), tuple(outs), *ins)
            if want_hout:
                t, xw, hsum = res
            else:
                t, xw = res
        else:
            n_out = params["post_W"].shape[1]
            out = _tc(
                _final_body, grid,
                [agg_spec, row(), dinv_spec, v_spec, v_spec, v_spec,
                 pl.BlockSpec((BS, 1), lambda i: (i, 0)),
                 _full_spec((D, n_out)), _full_spec((n_out,))],
                pl.BlockSpec((NUM_GRAPHS, n_out), lambda i: (0, 0)),
                jax.ShapeDtypeStruct((NUM_GRAPHS, n_out), jnp.float32),
                agg, xw, dinv, c["b"], c["g"], c["beta"], batch2,
                params["post_W"], params["post_b"],
                scratch_shapes=[pltpu.VMEM((NUM_GRAPHS, D), jnp.float32)])
    return out


# speedup vs baseline: 3.0999x; 1.0178x over previous
import functools

import jax
import jax.numpy as jnp
from jax import lax
from jax.experimental import pallas as pl
from jax.experimental.pallas import tpu as pltpu
from jax.experimental.pallas import tpu_sc as plsc

NC = 2
NS = 16
NW = NC * NS
CH = 80
NUM_GRAPHS = 128

_HI = jax.lax.Precision.HIGHEST


def _make_deg_kernel(E, N):
    epw = E // NW
    mesh = plsc.VectorSubcoreMesh(core_axis_name="c", subcore_axis_name="s")

    @functools.partial(
        pl.kernel,
        out_type=jax.ShapeDtypeStruct((NW * N,), jnp.float32),
        mesh=mesh,
        compiler_params=pltpu.CompilerParams(needs_layout_passes=False),
        scratch_types=[
            pltpu.VMEM((epw,), jnp.int32),
            pltpu.VMEM((N,), jnp.float32),
        ],
    )
    def deg_kernel(dst_hbm, out_hbm, dbuf, hist):
        c = lax.axis_index("c")
        s = lax.axis_index("s")
        wid = c * NS + s
        pltpu.sync_copy(dst_hbm.at[pl.ds(wid * epw, epw)], dbuf)

        zeros16 = jnp.zeros((16,), jnp.float32)

        def zbody(i, carry):
            hist[pl.ds(i * 16, 16)] = zeros16
            return carry

        lax.fori_loop(0, N // 16, zbody, 0)

        ones16 = jnp.full((16,), 1.0, jnp.float32)

        def body(i, carry):
            idx = dbuf[pl.ds(i * 16, 16)]
            plsc.addupdate_scatter(hist, [idx], ones16)
            return carry

        lax.fori_loop(0, epw // 16, body, 0)
        pltpu.sync_copy(hist, out_hbm.at[pl.ds(wid * N, N)])

    return deg_kernel


def _make_edge_kernel(E, N, D, nch, ch):
    nph = 5
    pch = nch // nph
    assert nch == nph * pch and pch % 2 == 0 and pch % 8 == 0
    rpw = (N // NS) // 8 * 8
    tail = N - NS * rpw
    mesh = plsc.VectorSubcoreMesh(core_axis_name="c", subcore_axis_name="s")

    @functools.partial(
        pl.kernel,
        out_type=jax.ShapeDtypeStruct((NC, N, D), jnp.float32),
        mesh=mesh,
        scratch_types=[
            [pltpu.VMEM((pch, ch), jnp.int32) for _ in range(2)],
            [pltpu.VMEM((pch, ch), jnp.int32) for _ in range(2)],
            pltpu.VMEM((ch, D), jnp.float32),
            pltpu.VMEM((ch, D), jnp.float32),
            pltpu.VMEM_SHARED((N + 8, D), jnp.float32),
            pltpu.SemaphoreType.DMA,
            pltpu.SemaphoreType.DMA,
            pltpu.SemaphoreType.DMA,
            pltpu.SemaphoreType.DMA,
        ],
    )
    def edge_kernel(t_hbm, src_hbm, dst_hbm, z_hbm, out_hbm,
                    sidxs, didxs, rows0, rows1, acc, sem0, sem1, isem, zsem):
        c = lax.axis_index("c")
        s = lax.axis_index("s")
        wid = c * NS + s
        row0 = s * rpw
        n2 = pch // 2

        def stage(q):
            pltpu.async_copy(src_hbm.at[wid, pl.ds(q * pch, pch)],
                             sidxs[q % 2], isem)
            pltpu.async_copy(dst_hbm.at[wid, pl.ds(q * pch, pch)],
                             didxs[q % 2], isem)

        def wait_stage(q):
            pltpu.make_async_copy(src_hbm.at[wid, pl.ds(q * pch, pch)],
                                  sidxs[q % 2], isem).wait()
            pltpu.make_async_copy(dst_hbm.at[wid, pl.ds(q * pch, pch)],
                                  didxs[q % 2], isem).wait()

        pltpu.async_copy(z_hbm.at[pl.ds(row0, rpw)],
                         acc.at[pl.ds(row0, rpw)], zsem)
        if tail:
            @pl.when(s == NS - 1)
            def _():
                pltpu.async_copy(z_hbm.at[pl.ds(NS * rpw, tail)],
                                 acc.at[pl.ds(NS * rpw, tail)], zsem)
        stage(0)
        wait_stage(0)
        pltpu.async_copy(t_hbm.at[sidxs[0].at[0]], rows0, sem0)
        pltpu.async_copy(t_hbm.at[sidxs[0].at[1]], rows1, sem1)
        pltpu.make_async_copy(z_hbm.at[pl.ds(row0, rpw)],
                              acc.at[pl.ds(row0, rpw)], zsem).wait()
        if tail:
            @pl.when(s == NS - 1)
            def _():
                pltpu.make_async_copy(z_hbm.at[pl.ds(NS * rpw, tail)],
                                      acc.at[pl.ds(NS * rpw, tail)],
                                      zsem).wait()
        plsc.subcore_barrier()

        for q in range(nph):
            sidx, didx = sidxs[q % 2], didxs[q % 2]
            nsidx = sidxs[(q + 1) % 2]
            if q + 1 < nph:
                stage(q + 1)

            def body(jj, carry):
                j = jj * 2
                pltpu.make_async_copy(t_hbm.at[sidx.at[j]], rows0,
                                      sem0).wait()
                pltpu.sync_copy(rows0, acc.at[didx.at[j]], add=True)

                @pl.when(jj + 1 < n2)
                def _():
                    pltpu.async_copy(t_hbm.at[sidx.at[j + 2]], rows0, sem0)

                if q + 1 < nph:
                    @pl.when(jj + 1 == n2)
                    def _():
                        wait_stage(q + 1)
                        pltpu.async_copy(t_hbm.at[nsidx.at[0]], rows0, sem0)

                pltpu.make_async_copy(t_hbm.at[sidx.at[j + 1]], rows1,
                                      sem1).wait()
                pltpu.sync_copy(rows1, acc.at[didx.at[j + 1]], add=True)

                @pl.when(jj + 1 < n2)
                def _():
                    pltpu.async_copy(t_hbm.at[sidx.at[j + 3]], rows1, sem1)

                if q + 1 < nph:
                    @pl.when(jj + 1 == n2)
                    def _():
                        pltpu.async_copy(t_hbm.at[nsidx.at[1]], rows1, sem1)

                return carry

            lax.fori_loop(0, n2, body, 0)
        plsc.subcore_barrier()
        pltpu.sync_copy(acc.at[pl.ds(row0, rpw)],
                        out_hbm.at[c, pl.ds(row0, rpw)])
        if tail:
            @pl.when(s == NS - 1)
            def _():
                pltpu.sync_copy(acc.at[pl.ds(NS * rpw, tail)],
                                out_hbm.at[c, pl.ds(NS * rpw, tail)])

    return edge_kernel


BS = 2000

def _row_spec():
    return pl.BlockSpec((BS, 128), lambda i: (i, 0))


def _full_spec(shape):
    nd = len(shape)
    return pl.BlockSpec(shape, lambda i: (0,) * nd)


def _pre_body(x_ref, pw_ref, pb_ref, w0_ref, degp_ref, dinv_ref, t0_ref):
    h = jnp.dot(x_ref[...], pw_ref[...], precision=_HI) + pb_ref[...]
    xw0 = jnp.dot(h, w0_ref[...], precision=_HI)
    deg = jnp.sum(degp_ref[...], axis=0) + 1.0
    dinv = lax.rsqrt(deg)[:, None]
    dinv_ref[...] = dinv
    t0_ref[...] = dinv * xw0


def _layer_norm_relu(pre, g, beta):
    mu = jnp.mean(pre, axis=-1, keepdims=True)
    var = jnp.mean(jnp.square(pre - mu), axis=-1, keepdims=True)
    o = (pre - mu) * lax.rsqrt(var + 1e-5) * g + beta
    return jnp.maximum(o, 0.0)


def _make_cell_body(has_hsum_in, want_hsum_out):
    def body(*refs):
        (agg_ref, t_in_ref, dinv_ref, b_ref, g_ref, beta_ref) = refs[:6]
        pos = 6
        if has_hsum_in:
            hin_ref = refs[pos]
            pos += 1
        wn_ref = refs[pos]
        pos += 1
        t_ref = refs[pos]
        pos += 1
        if want_hsum_out:
            hout_ref = refs[pos]

        dinv = dinv_ref[...]
        agg = agg_ref[...]
        pre = dinv * (agg[0] + agg[1] + t_in_ref[...]) + b_ref[...]
        o = _layer_norm_relu(pre, g_ref[...], beta_ref[...])
        if has_hsum_in:
            hsum = hin_ref[...] + o
        else:
            hsum = o
        if want_hsum_out:
            hout_ref[...] = hsum
        xwn = jnp.dot(hsum, wn_ref[...], precision=_HI)
        t_ref[...] = dinv * xwn

    return body


def _final_body(agg_ref, t_in_ref, dinv_ref, b_ref, g_ref, beta_ref,
                batch_ref, pw_ref, pb_ref, out_ref, pool_acc):
    i = pl.program_id(0)
    dinv = dinv_ref[...]
    agg = agg_ref[...]
    pre = dinv * (agg[0] + agg[1] + t_in_ref[...]) + b_ref[...]
    o = _layer_norm_relu(pre, g_ref[...], beta_ref[...])
    gid = lax.broadcasted_iota(jnp.int32, (1, NUM_GRAPHS), 1)
    onehot = (batch_ref[...] == gid).astype(jnp.float32)
    pooled = lax.dot_general(onehot, o, (((0,), (0,)), ((), ())),
                             precision=_HI)

    @pl.when(i == 0)
    def _():
        pool_acc[...] = jnp.zeros_like(pool_acc)

    pool_acc[...] += pooled

    @pl.when(i == pl.num_programs(0) - 1)
    def _():
        out_ref[...] = (jnp.dot(pool_acc[...], pw_ref[...], precision=_HI)
                        + pb_ref[...])


def _tc(body, grid, in_specs, out_specs, out_shape, *args, scratch_shapes=()):
    return pl.pallas_call(
        body, grid=grid, in_specs=in_specs, out_specs=out_specs,
        out_shape=out_shape, scratch_shapes=scratch_shapes,
        compiler_params=pltpu.CompilerParams(
            vmem_limit_bytes=60 * 1024 * 1024),
    )(*args)


def kernel(x, edge_index, batch, params):
    N, D = x.shape
    E = edge_index.shape[1]
    ch = 125
    epw = E // NW
    nch = -(-epw // (ch * 16)) * 16
    pad = nch * ch - epw
    src = jnp.pad(edge_index[0].reshape(NW, epw),
                  ((0, 0), (0, pad))).reshape(NW, nch, ch)
    dst_flat = edge_index[1]
    dst = jnp.pad(dst_flat.reshape(NW, epw), ((0, 0), (0, pad)),
                  constant_values=N).reshape(NW, nch, ch)
    zeros_nd = jnp.zeros((N, D), jnp.float32)
    batch2 = batch.reshape(N, 1)
    cells = params["cells"]
    ncells = len(cells)

    deg_kernel = _make_deg_kernel(E, N)
    edge_kernel = _make_edge_kernel(E, N, D, nch, ch)

    deg_p = deg_kernel(dst_flat).reshape(NW, N)
    grid = (N // BS,)
    row = _row_spec
    dinv_spec = pl.BlockSpec((BS, 1), lambda i: (i, 0))
    w_spec = _full_spec((D, D))
    v_spec = _full_spec((D,))
    agg_spec = pl.BlockSpec((NC, BS, D), lambda i: (0, i, 0))
    nd_sds = jax.ShapeDtypeStruct((N, D), jnp.float32)

    dinv, t = pl.pallas_call(
        _pre_body,
        out_shape=(jax.ShapeDtypeStruct((N, 1), jnp.float32), nd_sds),
        compiler_params=pltpu.CompilerParams(
            vmem_limit_bytes=60 * 1024 * 1024),
    )(x, params["pre_W"], params["pre_b"], cells[0]["W"], deg_p)

    hsum = None
    for i in range(ncells):
        agg = edge_kernel(t, src, dst, zeros_nd)
        c = cells[i]
        if i < ncells - 1:
            has_hin = i > 0
            want_hout = i < ncells - 2
            ins = [agg, t, dinv, c["b"], c["g"], c["beta"]]
            specs = [agg_spec, row(), dinv_spec, v_spec, v_spec, v_spec]
            if has_hin:
                ins.append(hsum)
                specs.append(row())
            ins.append(cells[i + 1]["W"])
            specs.append(w_spec)
            outs = [nd_sds]
            out_specs = [row()]
            if want_hout:
                outs.append(nd_sds)
                out_specs.append(row())
            res = _tc(_make_cell_body(has_hin, want_hout), grid,
                      specs, tuple(out_specs), tuple(outs), *ins)
            if want_hout:
                t, hsum = res
            else:
                t, = res
        else:
            n_out = params["post_W"].shape[1]
            out = _tc(
                _final_body, grid,
                [agg_spec, row(), dinv_spec, v_spec, v_spec, v_spec,
                 pl.BlockSpec((BS, 1), lambda i: (i, 0)),
                 _full_spec((D, n_out)), _full_spec((n_out,))],
                pl.BlockSpec((NUM_GRAPHS, n_out), lambda i: (0, 0)),
                jax.ShapeDtypeStruct((NUM_GRAPHS, n_out), jnp.float32),
                agg, t, dinv, c["b"], c["g"], c["beta"], batch2,
                params["post_W"], params["post_b"],
                scratch_shapes=[pltpu.VMEM((NUM_GRAPHS, D), jnp.float32)])
    return out


# speedup vs baseline: 3.1968x; 1.0313x over previous
import functools

import jax
import jax.numpy as jnp
from jax import lax
from jax.experimental import pallas as pl
from jax.experimental.pallas import tpu as pltpu
from jax.experimental.pallas import tpu_sc as plsc

NC = 2
NS = 16
NW = NC * NS
CH = 80
NUM_GRAPHS = 128

_HI = jax.lax.Precision.HIGHEST


def _make_deg_kernel(E, N):
    epw = E // NW
    mesh = plsc.VectorSubcoreMesh(core_axis_name="c", subcore_axis_name="s")

    @functools.partial(
        pl.kernel,
        out_type=jax.ShapeDtypeStruct((NW * N,), jnp.float32),
        mesh=mesh,
        compiler_params=pltpu.CompilerParams(needs_layout_passes=False),
        scratch_types=[
            pltpu.VMEM((epw,), jnp.int32),
            pltpu.VMEM((N,), jnp.float32),
        ],
    )
    def deg_kernel(dst_hbm, out_hbm, dbuf, hist):
        c = lax.axis_index("c")
        s = lax.axis_index("s")
        wid = c * NS + s
        pltpu.sync_copy(dst_hbm.at[pl.ds(wid * epw, epw)], dbuf)

        zeros16 = jnp.zeros((16,), jnp.float32)

        def zbody(i, carry):
            hist[pl.ds(i * 16, 16)] = zeros16
            return carry

        lax.fori_loop(0, N // 16, zbody, 0)

        ones16 = jnp.full((16,), 1.0, jnp.float32)

        def body(i, carry):
            idx = dbuf[pl.ds(i * 16, 16)]
            plsc.addupdate_scatter(hist, [idx], ones16)
            return carry

        lax.fori_loop(0, epw // 16, body, 0)
        pltpu.sync_copy(hist, out_hbm.at[pl.ds(wid * N, N)])

    return deg_kernel


def _make_edge_kernel(E, N, D, nch, ch):
    nph = 5
    pch = nch // nph
    assert nch == nph * pch and pch % 2 == 0 and pch % 8 == 0
    rpw = (N // NS) // 8 * 8
    tail = N - NS * rpw
    zch = 104
    assert rpw % zch == 0 and zch <= ch and tail <= zch
    mesh = plsc.VectorSubcoreMesh(core_axis_name="c", subcore_axis_name="s")

    @functools.partial(
        pl.kernel,
        out_type=jax.ShapeDtypeStruct((NC, N, D), jnp.float32),
        mesh=mesh,
        scratch_types=[
            [pltpu.VMEM((pch, ch), jnp.int32) for _ in range(2)],
            [pltpu.VMEM((pch, ch), jnp.int32) for _ in range(2)],
            pltpu.VMEM((ch, D), jnp.float32),
            pltpu.VMEM((ch, D), jnp.float32),
            pltpu.VMEM_SHARED((N + 8, D), jnp.float32),
            pltpu.SemaphoreType.DMA,
            pltpu.SemaphoreType.DMA,
            pltpu.SemaphoreType.DMA,
            pltpu.SemaphoreType.DMA,
        ],
    )
    def edge_kernel(t_hbm, src_hbm, dst_hbm, out_hbm,
                    sidxs, didxs, rows0, rows1, acc, sem0, sem1, isem, zsem):
        c = lax.axis_index("c")
        s = lax.axis_index("s")
        wid = c * NS + s
        row0 = s * rpw
        n2 = pch // 2

        def stage(q):
            pltpu.async_copy(src_hbm.at[wid, pl.ds(q * pch, pch)],
                             sidxs[q % 2], isem)
            pltpu.async_copy(dst_hbm.at[wid, pl.ds(q * pch, pch)],
                             didxs[q % 2], isem)

        def wait_stage(q):
            pltpu.make_async_copy(src_hbm.at[wid, pl.ds(q * pch, pch)],
                                  sidxs[q % 2], isem).wait()
            pltpu.make_async_copy(dst_hbm.at[wid, pl.ds(q * pch, pch)],
                                  didxs[q % 2], isem).wait()

        stage(0)
        zeros16 = jnp.zeros((16,), jnp.float32)

        def zrow(i, carry):
            def zcol(k, carry2):
                rows1[i, pl.ds(k * 16, 16)] = zeros16
                return carry2

            return lax.fori_loop(0, D // 16, zcol, carry)

        lax.fori_loop(0, zch, zrow, 0)
        for k in range(rpw // zch):
            pltpu.async_copy(rows1.at[pl.ds(0, zch)],
                             acc.at[pl.ds(row0 + k * zch, zch)], zsem)
        if tail:
            @pl.when(s == NS - 1)
            def _():
                pltpu.async_copy(rows1.at[pl.ds(0, tail)],
                                 acc.at[pl.ds(NS * rpw, tail)], zsem)
        wait_stage(0)
        pltpu.async_copy(t_hbm.at[sidxs[0].at[0]], rows0, sem0)
        for k in range(rpw // zch):
            pltpu.make_async_copy(rows1.at[pl.ds(0, zch)],
                                  acc.at[pl.ds(row0 + k * zch, zch)],
                                  zsem).wait()
        if tail:
            @pl.when(s == NS - 1)
            def _():
                pltpu.make_async_copy(rows1.at[pl.ds(0, tail)],
                                      acc.at[pl.ds(NS * rpw, tail)],
                                      zsem).wait()
        pltpu.async_copy(t_hbm.at[sidxs[0].at[1]], rows1, sem1)
        plsc.subcore_barrier()

        for q in range(nph):
            sidx, didx = sidxs[q % 2], didxs[q % 2]
            nsidx = sidxs[(q + 1) % 2]
            if q + 1 < nph:
                stage(q + 1)

            def body(jj, carry):
                j = jj * 2
                pltpu.make_async_copy(t_hbm.at[sidx.at[j]], rows0,
                                      sem0).wait()
                pltpu.sync_copy(rows0, acc.at[didx.at[j]], add=True)

                @pl.when(jj + 1 < n2)
                def _():
                    pltpu.async_copy(t_hbm.at[sidx.at[j + 2]], rows0, sem0)

                if q + 1 < nph:
                    @pl.when(jj + 1 == n2)
                    def _():
                        wait_stage(q + 1)
                        pltpu.async_copy(t_hbm.at[nsidx.at[0]], rows0, sem0)

                pltpu.make_async_copy(t_hbm.at[sidx.at[j + 1]], rows1,
                                      sem1).wait()
                pltpu.sync_copy(rows1, acc.at[didx.at[j + 1]], add=True)

                @pl.when(jj + 1 < n2)
                def _():
                    pltpu.async_copy(t_hbm.at[sidx.at[j + 3]], rows1, sem1)

                if q + 1 < nph:
                    @pl.when(jj + 1 == n2)
                    def _():
                        pltpu.async_copy(t_hbm.at[nsidx.at[1]], rows1, sem1)

                return carry

            lax.fori_loop(0, n2, body, 0)
        plsc.subcore_barrier()
        pltpu.sync_copy(acc.at[pl.ds(row0, rpw)],
                        out_hbm.at[c, pl.ds(row0, rpw)])
        if tail:
            @pl.when(s == NS - 1)
            def _():
                pltpu.sync_copy(acc.at[pl.ds(NS * rpw, tail)],
                                out_hbm.at[c, pl.ds(NS * rpw, tail)])

    return edge_kernel


BS = 2000

def _row_spec():
    return pl.BlockSpec((BS, 128), lambda i: (i, 0))


def _full_spec(shape):
    nd = len(shape)
    return pl.BlockSpec(shape, lambda i: (0,) * nd)


def _pre_body(x_ref, pw_ref, pb_ref, w0_ref, degp_ref, dinv_ref, t0_ref):
    h = jnp.dot(x_ref[...], pw_ref[...], precision=_HI) + pb_ref[...]
    xw0 = jnp.dot(h, w0_ref[...], precision=_HI)
    deg = jnp.sum(degp_ref[...], axis=0) + 1.0
    dinv = lax.rsqrt(deg)[:, None]
    dinv_ref[...] = dinv
    t0_ref[...] = dinv * xw0


def _layer_norm_relu(pre, g, beta):
    mu = jnp.mean(pre, axis=-1, keepdims=True)
    var = jnp.mean(jnp.square(pre - mu), axis=-1, keepdims=True)
    o = (pre - mu) * lax.rsqrt(var + 1e-5) * g + beta
    return jnp.maximum(o, 0.0)


def _make_cell_body(has_hsum_in, want_hsum_out):
    def body(*refs):
        (agg_ref, t_in_ref, dinv_ref, b_ref, g_ref, beta_ref) = refs[:6]
        pos = 6
        if has_hsum_in:
            hin_ref = refs[pos]
            pos += 1
        wn_ref = refs[pos]
        pos += 1
        t_ref = refs[pos]
        pos += 1
        if want_hsum_out:
            hout_ref = refs[pos]

        dinv = dinv_ref[...]
        agg = agg_ref[...]
        pre = dinv * (agg[0] + agg[1] + t_in_ref[...]) + b_ref[...]
        o = _layer_norm_relu(pre, g_ref[...], beta_ref[...])
        if has_hsum_in:
            hsum = hin_ref[...] + o
        else:
            hsum = o
        if want_hsum_out:
            hout_ref[...] = hsum
        xwn = jnp.dot(hsum, wn_ref[...], precision=_HI)
        t_ref[...] = dinv * xwn

    return body


def _final_body(agg_ref, t_in_ref, dinv_ref, b_ref, g_ref, beta_ref,
                batch_ref, pw_ref, pb_ref, out_ref, pool_acc):
    i = pl.program_id(0)
    dinv = dinv_ref[...]
    agg = agg_ref[...]
    pre = dinv * (agg[0] + agg[1] + t_in_ref[...]) + b_ref[...]
    o = _layer_norm_relu(pre, g_ref[...], beta_ref[...])
    gid = lax.broadcasted_iota(jnp.int32, (1, NUM_GRAPHS), 1)
    onehot = (batch_ref[...] == gid).astype(jnp.float32)
    pooled = lax.dot_general(onehot, o, (((0,), (0,)), ((), ())),
                             precision=_HI)

    @pl.when(i == 0)
    def _():
        pool_acc[...] = jnp.zeros_like(pool_acc)

    pool_acc[...] += pooled

    @pl.when(i == pl.num_programs(0) - 1)
    def _():
        out_ref[...] = (jnp.dot(pool_acc[...], pw_ref[...], precision=_HI)
                        + pb_ref[...])


def _tc(body, grid, in_specs, out_specs, out_shape, *args, scratch_shapes=()):
    return pl.pallas_call(
        body, grid=grid, in_specs=in_specs, out_specs=out_specs,
        out_shape=out_shape, scratch_shapes=scratch_shapes,
        compiler_params=pltpu.CompilerParams(
            vmem_limit_bytes=60 * 1024 * 1024),
    )(*args)


def kernel(x, edge_index, batch, params):
    N, D = x.shape
    E = edge_index.shape[1]
    ch = 125
    epw = E // NW
    nch = -(-epw // (ch * 16)) * 16
    pad = nch * ch - epw
    src = jnp.pad(edge_index[0].reshape(NW, epw),
                  ((0, 0), (0, pad))).reshape(NW, nch, ch)
    dst_flat = edge_index[1]
    dst = jnp.pad(dst_flat.reshape(NW, epw), ((0, 0), (0, pad)),
                  constant_values=N).reshape(NW, nch, ch)
    batch2 = batch.reshape(N, 1)
    cells = params["cells"]
    ncells = len(cells)

    deg_kernel = _make_deg_kernel(E, N)
    edge_kernel = _make_edge_kernel(E, N, D, nch, ch)

    deg_p = deg_kernel(dst_flat).reshape(NW, N)
    grid = (N // BS,)
    row = _row_spec
    dinv_spec = pl.BlockSpec((BS, 1), lambda i: (i, 0))
    w_spec = _full_spec((D, D))
    v_spec = _full_spec((D,))
    agg_spec = pl.BlockSpec((NC, BS, D), lambda i: (0, i, 0))
    nd_sds = jax.ShapeDtypeStruct((N, D), jnp.float32)

    dinv, t = pl.pallas_call(
        _pre_body,
        out_shape=(jax.ShapeDtypeStruct((N, 1), jnp.float32), nd_sds),
        compiler_params=pltpu.CompilerParams(
            vmem_limit_bytes=60 * 1024 * 1024),
    )(x, params["pre_W"], params["pre_b"], cells[0]["W"], deg_p)

    hsum = None
    for i in range(ncells):
        agg = edge_kernel(t, src, dst)
        c = cells[i]
        if i < ncells - 1:
            has_hin = i > 0
            want_hout = i < ncells - 2
            ins = [agg, t, dinv, c["b"], c["g"], c["beta"]]
            specs = [agg_spec, row(), dinv_spec, v_spec, v_spec, v_spec]
            if has_hin:
                ins.append(hsum)
                specs.append(row())
            ins.append(cells[i + 1]["W"])
            specs.append(w_spec)
            outs = [nd_sds]
            out_specs = [row()]
            if want_hout:
                outs.append(nd_sds)
                out_specs.append(row())
            res = _tc(_make_cell_body(has_hin, want_hout), grid,
                      specs, tuple(out_specs), tuple(outs), *ins)
            if want_hout:
                t, hsum = res
            else:
                t, = res
        else:
            n_out = params["post_W"].shape[1]
            out = _tc(
                _final_body, grid,
                [agg_spec, row(), dinv_spec, v_spec, v_spec, v_spec,
                 pl.BlockSpec((BS, 1), lambda i: (i, 0)),
                 _full_spec((D, n_out)), _full_spec((n_out,))],
                pl.BlockSpec((NUM_GRAPHS, n_out), lambda i: (0, 0)),
                jax.ShapeDtypeStruct((NUM_GRAPHS, n_out), jnp.float32),
                agg, t, dinv, c["b"], c["g"], c["beta"], batch2,
                params["post_W"], params["post_b"],
                scratch_shapes=[pltpu.VMEM((NUM_GRAPHS, D), jnp.float32)])
    return out


# speedup vs baseline: 3.1976x; 1.0002x over previous
import functools

import jax
import jax.numpy as jnp
from jax import lax
from jax.experimental import pallas as pl
from jax.experimental.pallas import tpu as pltpu
from jax.experimental.pallas import tpu_sc as plsc

NC = 2
NS = 16
NW = NC * NS
CH = 80
NUM_GRAPHS = 128

_HI = jax.lax.Precision.HIGHEST


def _make_deg_kernel(E, N):
    epw = E // NW
    mesh = plsc.VectorSubcoreMesh(core_axis_name="c", subcore_axis_name="s")

    @functools.partial(
        pl.kernel,
        out_type=jax.ShapeDtypeStruct((NW * N,), jnp.float32),
        mesh=mesh,
        compiler_params=pltpu.CompilerParams(needs_layout_passes=False),
        scratch_types=[
            pltpu.VMEM((epw,), jnp.int32),
            pltpu.VMEM((N,), jnp.float32),
            pltpu.SemaphoreType.DMA,
        ],
    )
    def deg_kernel(dst_hbm, out_hbm, dbuf, hist, sem):
        c = lax.axis_index("c")
        s = lax.axis_index("s")
        wid = c * NS + s
        pltpu.async_copy(dst_hbm.at[pl.ds(wid * epw, epw)], dbuf, sem)

        zeros16 = jnp.zeros((16,), jnp.float32)

        def zbody(i, carry):
            hist[pl.ds(i * 16, 16)] = zeros16
            return carry

        lax.fori_loop(0, N // 16, zbody, 0)
        pltpu.make_async_copy(dst_hbm.at[pl.ds(wid * epw, epw)], dbuf,
                              sem).wait()

        ones16 = jnp.full((16,), 1.0, jnp.float32)

        def body(i, carry):
            idx = dbuf[pl.ds(i * 16, 16)]
            plsc.addupdate_scatter(hist, [idx], ones16)
            return carry

        lax.fori_loop(0, epw // 16, body, 0)
        pltpu.sync_copy(hist, out_hbm.at[pl.ds(wid * N, N)])

    return deg_kernel


def _make_edge_kernel(E, N, D, nch, ch):
    nph = 5
    pch = nch // nph
    assert nch == nph * pch and pch % 2 == 0 and pch % 8 == 0
    rpw = (N // NS) // 8 * 8
    tail = N - NS * rpw
    zch = 104
    assert rpw % zch == 0 and zch <= ch and tail <= zch
    mesh = plsc.VectorSubcoreMesh(core_axis_name="c", subcore_axis_name="s")

    @functools.partial(
        pl.kernel,
        out_type=jax.ShapeDtypeStruct((NC, N, D), jnp.float32),
        mesh=mesh,
        scratch_types=[
            [pltpu.VMEM((pch, ch), jnp.int32) for _ in range(2)],
            [pltpu.VMEM((pch, ch), jnp.int32) for _ in range(2)],
            pltpu.VMEM((ch, D), jnp.float32),
            pltpu.VMEM((ch, D), jnp.float32),
            pltpu.VMEM_SHARED((N + 8, D), jnp.float32),
            pltpu.SemaphoreType.DMA,
            pltpu.SemaphoreType.DMA,
            pltpu.SemaphoreType.DMA,
            pltpu.SemaphoreType.DMA,
        ],
    )
    def edge_kernel(t_hbm, src_hbm, dst_hbm, out_hbm,
                    sidxs, didxs, rows0, rows1, acc, sem0, sem1, isem, zsem):
        c = lax.axis_index("c")
        s = lax.axis_index("s")
        wid = c * NS + s
        row0 = s * rpw
        n2 = pch // 2

        def stage(q):
            pltpu.async_copy(src_hbm.at[wid, pl.ds(q * pch, pch)],
                             sidxs[q % 2], isem)
            pltpu.async_copy(dst_hbm.at[wid, pl.ds(q * pch, pch)],
                             didxs[q % 2], isem)

        def wait_stage(q):
            pltpu.make_async_copy(src_hbm.at[wid, pl.ds(q * pch, pch)],
                                  sidxs[q % 2], isem).wait()
            pltpu.make_async_copy(dst_hbm.at[wid, pl.ds(q * pch, pch)],
                                  didxs[q % 2], isem).wait()

        stage(0)
        zeros16 = jnp.zeros((16,), jnp.float32)

        def zrow(i, carry):
            def zcol(k, carry2):
                rows1[i, pl.ds(k * 16, 16)] = zeros16
                return carry2

            return lax.fori_loop(0, D // 16, zcol, carry)

        lax.fori_loop(0, zch, zrow, 0)
        for k in range(rpw // zch):
            pltpu.async_copy(rows1.at[pl.ds(0, zch)],
                             acc.at[pl.ds(row0 + k * zch, zch)], zsem)
        if tail:
            @pl.when(s == NS - 1)
            def _():
                pltpu.async_copy(rows1.at[pl.ds(0, tail)],
                                 acc.at[pl.ds(NS * rpw, tail)], zsem)
        wait_stage(0)
        pltpu.async_copy(t_hbm.at[sidxs[0].at[0]], rows0, sem0)
        for k in range(rpw // zch):
            pltpu.make_async_copy(rows1.at[pl.ds(0, zch)],
                                  acc.at[pl.ds(row0 + k * zch, zch)],
                                  zsem).wait()
        if tail:
            @pl.when(s == NS - 1)
            def _():
                pltpu.make_async_copy(rows1.at[pl.ds(0, tail)],
                                      acc.at[pl.ds(NS * rpw, tail)],
                                      zsem).wait()
        pltpu.async_copy(t_hbm.at[sidxs[0].at[1]], rows1, sem1)
        plsc.subcore_barrier()

        for q in range(nph):
            sidx, didx = sidxs[q % 2], didxs[q % 2]
            nsidx = sidxs[(q + 1) % 2]
            if q + 1 < nph:
                stage(q + 1)

            def body(jj, carry):
                j = jj * 2
                pltpu.make_async_copy(t_hbm.at[sidx.at[j]], rows0,
                                      sem0).wait()
                pltpu.sync_copy(rows0, acc.at[didx.at[j]], add=True)

                @pl.when(jj + 1 < n2)
                def _():
                    pltpu.async_copy(t_hbm.at[sidx.at[j + 2]], rows0, sem0)

                if q + 1 < nph:
                    @pl.when(jj + 1 == n2)
                    def _():
                        wait_stage(q + 1)
                        pltpu.async_copy(t_hbm.at[nsidx.at[0]], rows0, sem0)

                pltpu.make_async_copy(t_hbm.at[sidx.at[j + 1]], rows1,
                                      sem1).wait()
                pltpu.sync_copy(rows1, acc.at[didx.at[j + 1]], add=True)

                @pl.when(jj + 1 < n2)
                def _():
                    pltpu.async_copy(t_hbm.at[sidx.at[j + 3]], rows1, sem1)

                if q + 1 < nph:
                    @pl.when(jj + 1 == n2)
                    def _():
                        pltpu.async_copy(t_hbm.at[nsidx.at[1]], rows1, sem1)

                return carry

            lax.fori_loop(0, n2, body, 0)
        plsc.subcore_barrier()
        pltpu.sync_copy(acc.at[pl.ds(row0, rpw)],
                        out_hbm.at[c, pl.ds(row0, rpw)])
        if tail:
            @pl.when(s == NS - 1)
            def _():
                pltpu.sync_copy(acc.at[pl.ds(NS * rpw, tail)],
                                out_hbm.at[c, pl.ds(NS * rpw, tail)])

    return edge_kernel


BS = 2000

def _row_spec():
    return pl.BlockSpec((BS, 128), lambda i: (i, 0))


def _full_spec(shape):
    nd = len(shape)
    return pl.BlockSpec(shape, lambda i: (0,) * nd)


def _pre_body(x_ref, pw_ref, pb_ref, w0_ref, degp_ref, dinv_ref, t0_ref):
    h = jnp.dot(x_ref[...], pw_ref[...], precision=_HI) + pb_ref[...]
    xw0 = jnp.dot(h, w0_ref[...], precision=_HI)
    deg = jnp.sum(degp_ref[...], axis=0) + 1.0
    dinv = lax.rsqrt(deg)[:, None]
    dinv_ref[...] = dinv
    t0_ref[...] = dinv * xw0


def _layer_norm_relu(pre, g, beta):
    mu = jnp.mean(pre, axis=-1, keepdims=True)
    var = jnp.mean(jnp.square(pre - mu), axis=-1, keepdims=True)
    o = (pre - mu) * lax.rsqrt(var + 1e-5) * g + beta
    return jnp.maximum(o, 0.0)


def _make_cell_body(has_hsum_in, want_hsum_out):
    def body(*refs):
        (agg_ref, t_in_ref, dinv_ref, b_ref, g_ref, beta_ref) = refs[:6]
        pos = 6
        if has_hsum_in:
            hin_ref = refs[pos]
            pos += 1
        wn_ref = refs[pos]
        pos += 1
        t_ref = refs[pos]
        pos += 1
        if want_hsum_out:
            hout_ref = refs[pos]

        dinv = dinv_ref[...]
        agg = agg_ref[...]
        pre = dinv * (agg[0] + agg[1] + t_in_ref[...]) + b_ref[...]
        o = _layer_norm_relu(pre, g_ref[...], beta_ref[...])
        if has_hsum_in:
            hsum = hin_ref[...] + o
        else:
            hsum = o
        if want_hsum_out:
            hout_ref[...] = hsum
        xwn = jnp.dot(hsum, wn_ref[...], precision=_HI)
        t_ref[...] = dinv * xwn

    return body


def _final_body(agg_ref, t_in_ref, dinv_ref, b_ref, g_ref, beta_ref,
                batch_ref, pw_ref, pb_ref, out_ref, pool_acc):
    i = pl.program_id(0)
    dinv = dinv_ref[...]
    agg = agg_ref[...]
    pre = dinv * (agg[0] + agg[1] + t_in_ref[...]) + b_ref[...]
    o = _layer_norm_relu(pre, g_ref[...], beta_ref[...])
    gid = lax.broadcasted_iota(jnp.int32, (1, NUM_GRAPHS), 1)
    onehot = (batch_ref[...] == gid).astype(jnp.float32)
    pooled = lax.dot_general(onehot, o, (((0,), (0,)), ((), ())),
                             precision=_HI)

    @pl.when(i == 0)
    def _():
        pool_acc[...] = jnp.zeros_like(pool_acc)

    pool_acc[...] += pooled

    @pl.when(i == pl.num_programs(0) - 1)
    def _():
        out_ref[...] = (jnp.dot(pool_acc[...], pw_ref[...], precision=_HI)
                        + pb_ref[...])


def _tc(body, grid, in_specs, out_specs, out_shape, *args, scratch_shapes=()):
    return pl.pallas_call(
        body, grid=grid, in_specs=in_specs, out_specs=out_specs,
        out_shape=out_shape, scratch_shapes=scratch_shapes,
        compiler_params=pltpu.CompilerParams(
            vmem_limit_bytes=60 * 1024 * 1024),
    )(*args)


def kernel(x, edge_index, batch, params):
    N, D = x.shape
    E = edge_index.shape[1]
    ch = 125
    epw = E // NW
    nch = -(-epw // (ch * 16)) * 16
    pad = nch * ch - epw
    src = jnp.pad(edge_index[0].reshape(NW, epw),
                  ((0, 0), (0, pad))).reshape(NW, nch, ch)
    dst_flat = edge_index[1]
    dst = jnp.pad(dst_flat.reshape(NW, epw), ((0, 0), (0, pad)),
                  constant_values=N).reshape(NW, nch, ch)
    batch2 = batch.reshape(N, 1)
    cells = params["cells"]
    ncells = len(cells)

    deg_kernel = _make_deg_kernel(E, N)
    edge_kernel = _make_edge_kernel(E, N, D, nch, ch)

    deg_p = deg_kernel(dst_flat).reshape(NW, N)
    grid = (N // BS,)
    row = _row_spec
    dinv_spec = pl.BlockSpec((BS, 1), lambda i: (i, 0))
    w_spec = _full_spec((D, D))
    v_spec = _full_spec((D,))
    agg_spec = pl.BlockSpec((NC, BS, D), lambda i: (0, i, 0))
    nd_sds = jax.ShapeDtypeStruct((N, D), jnp.float32)

    dinv, t = pl.pallas_call(
        _pre_body,
        out_shape=(jax.ShapeDtypeStruct((N, 1), jnp.float32), nd_sds),
        compiler_params=pltpu.CompilerParams(
            vmem_limit_bytes=60 * 1024 * 1024),
    )(x, params["pre_W"], params["pre_b"], cells[0]["W"], deg_p)

    hsum = None
    for i in range(ncells):
        agg = edge_kernel(t, src, dst)
        c = cells[i]
        if i < ncells - 1:
            has_hin = i > 0
            want_hout = i < ncells - 2
            ins = [agg, t, dinv, c["b"], c["g"], c["beta"]]
            specs = [agg_spec, row(), dinv_spec, v_spec, v_spec, v_spec]
            if has_hin:
                ins.append(hsum)
                specs.append(row())
            ins.append(cells[i + 1]["W"])
            specs.append(w_spec)
            outs = [nd_sds]
            out_specs = [row()]
            if want_hout:
                outs.append(nd_sds)
                out_specs.append(row())
            res = _tc(_make_cell_body(has_hin, want_hout), grid,
                      specs, tuple(out_specs), tuple(outs), *ins)
            if want_hout:
                t, hsum = res
            else:
                t, = res
        else:
            n_out = params["post_W"].shape[1]
            out = _tc(
                _final_body, grid,
                [agg_spec, row(), dinv_spec, v_spec, v_spec, v_spec,
                 pl.BlockSpec((BS, 1), lambda i: (i, 0)),
                 _full_spec((D, n_out)), _full_spec((n_out,))],
                pl.BlockSpec((NUM_GRAPHS, n_out), lambda i: (0, 0)),
                jax.ShapeDtypeStruct((NUM_GRAPHS, n_out), jnp.float32),
                agg, t, dinv, c["b"], c["g"], c["beta"], batch2,
                params["post_W"], params["post_b"],
                scratch_shapes=[pltpu.VMEM((NUM_GRAPHS, D), jnp.float32)])
    return out
